# baseline (speedup 1.0000x reference)
import math

import jax
import jax.numpy as jnp
from jax import lax
from jax.experimental import pallas as pl
from jax.experimental.pallas import tpu as pltpu
from jax.experimental.pallas import tpu_sc as plsc

VOCAB = 1000000
D = 64
L = 16
NC, NS = 2, 16
NW = NC * NS
BATCH = 4096
HIST = 200
BB = BATCH // NW
HB = 8

MAX_NORM = (1.0 - 0.001) / math.sqrt(1.0)
INV_MAX_NORM = 1.0 / MAX_NORM


def _rsqrt_nr(s):
    i = lax.bitcast_convert_type(s, jnp.int32)
    i = jnp.int32(0x5F3759DF) - lax.shift_right_arithmetic(i, 1)
    y = lax.bitcast_convert_type(i, jnp.float32)
    y = y * (1.5 - 0.5 * (s * y) * y)
    y = y * (1.5 - 0.5 * (s * y) * y)
    y = y * (1.5 - 0.5 * (s * y) * y)
    return y


def _recip_nr(d):
    i = lax.bitcast_convert_type(d, jnp.int32)
    z = lax.bitcast_convert_type(jnp.int32(0x7EF127EA) - i, jnp.float32)
    z = z * (2.0 - d * z)
    z = z * (2.0 - d * z)
    z = z * (2.0 - d * z)
    return z


def _body(idsT_hbm, wp_hbm, outT_hbm, ids_v, idx_v, rows_v, tile_v, sem):
    wid = lax.axis_index("s") * NC + lax.axis_index("c")
    b0 = wid * BB
    lane = lax.iota(jnp.int32, L)

    def hblk_body(hb, _):
        h0 = hb * HB
        pltpu.sync_copy(
            idsT_hbm.at[pl.ds(h0, HB), pl.ds(b0, BB)], ids_v
        )

        def h_body(hh, _):
            for k in range(BB // L):
                idv = ids_v[hh, pl.ds(k * L, L)]
                idx_v[pl.ds(k * L, L)] = lax.shift_right_logical(idv, 1)
            pltpu.async_copy(wp_hbm.at[idx_v], rows_v, sem).wait()

            def grp_body(g, _):
                idrow = ids_v[hh, pl.ds(g * L, L)]
                bases = (idrow & 1) * D
                for jj in range(L):
                    base = bases[jj]
                    j = g * L + jj
                    v = [
                        rows_v[j, pl.ds(base + k * L, L)]
                        for k in range(D // L)
                    ]
                    ss = jnp.zeros((L,), jnp.float32)
                    for k in range(D // L):
                        ss = ss + v[k] * v[k]
                    s = jnp.sum(ss)
                    rs = _rsqrt_nr(s)
                    norm = s * rs
                    scale = jnp.minimum(norm * INV_MAX_NORM, 1.0)
                    f = _recip_nr(scale + 1e-8)
                    jcol = jnp.full((L,), j, jnp.int32)
                    for k in range(D // L):
                        crow = k * L + lane
                        plsc.store_scatter(tile_v, [crow, jcol], v[k] * f)
                return 0

            lax.fori_loop(0, BB // L, grp_body, 0)
            pltpu.sync_copy(
                tile_v, outT_hbm.at[h0 + hh, :, pl.ds(b0, BB)]
            )
            return 0

        lax.fori_loop(0, HB, h_body, 0)
        return 0

    lax.fori_loop(0, HIST // HB, hblk_body, 0)


@jax.jit
def _run(idsT, wp):
    mesh = plsc.VectorSubcoreMesh(core_axis_name="c", subcore_axis_name="s")
    return pl.kernel(
        _body,
        out_type=jax.ShapeDtypeStruct((HIST, D, BATCH), jnp.float32),
        mesh=mesh,
        compiler_params=pltpu.CompilerParams(needs_layout_passes=False),
        scratch_types=[
            pltpu.VMEM((HB, BB), jnp.int32),
            pltpu.VMEM((BB,), jnp.int32),
            pltpu.VMEM((BB, 2 * D), jnp.float32),
            pltpu.VMEM((D, BB), jnp.float32),
            pltpu.SemaphoreType.DMA,
        ],
    )(idsT, wp)


def kernel(input_ids, weight):
    idsT = input_ids.T
    wp = weight.reshape(VOCAB // 2, 2 * D)
    outT = _run(idsT, wp)
    return outT.transpose(2, 0, 1)


# speedup vs baseline: 1.9409x; 1.9409x over previous
import math

import jax
import jax.numpy as jnp
from jax import lax
from jax.experimental import pallas as pl
from jax.experimental.pallas import tpu as pltpu
from jax.experimental.pallas import tpu_sc as plsc

VOCAB = 1000000
D = 64
L = 16
NC, NS = 2, 16
NW = NC * NS
BATCH = 4096
HIST = 200
BB = BATCH // NW

MAX_NORM = (1.0 - 0.001) / math.sqrt(1.0)
INV_MAX_NORM = 1.0 / MAX_NORM


def _rsqrt_nr(s):
    i = lax.bitcast_convert_type(s, jnp.int32)
    i = jnp.int32(0x5F3759DF) - lax.shift_right_arithmetic(i, 1)
    y = lax.bitcast_convert_type(i, jnp.float32)
    y = y * (1.5 - 0.5 * (s * y) * y)
    y = y * (1.5 - 0.5 * (s * y) * y)
    y = y * (1.5 - 0.5 * (s * y) * y)
    return y


def _recip_nr(d):
    i = lax.bitcast_convert_type(d, jnp.int32)
    z = lax.bitcast_convert_type(jnp.int32(0x7EF127EA) - i, jnp.float32)
    z = z * (2.0 - d * z)
    z = z * (2.0 - d * z)
    z = z * (2.0 - d * z)
    return z


def _factor(acc):
    rs = _rsqrt_nr(acc)
    norm = acc * rs
    scale = jnp.minimum(norm * INV_MAX_NORM, 1.0)
    return _recip_nr(scale + 1e-8)


def _body(
    idsT_hbm, wp_hbm, outT_hbm,
    ids_all, idx_all, rows0, rows1, tile0, tile1, rsem, osem,
):
    wid = lax.axis_index("s") * NC + lax.axis_index("c")
    b0 = wid * BB
    lane = lax.iota(jnp.int32, L)
    krow = [k * L + lane for k in range(D // L)]

    pltpu.sync_copy(idsT_hbm.at[:, pl.ds(b0, BB)], ids_all)

    def idx_body(h, _):
        for k in range(BB // L):
            idx_all[h, pl.ds(k * L, L)] = lax.shift_right_logical(
                ids_all[h, pl.ds(k * L, L)], 1
            )
        return 0

    lax.fori_loop(0, HIST, idx_body, 0)

    rows = (rows0, rows1)
    tiles = (tile0, tile1)

    def compute(h, rb, tb):
        def g_body(g, _):
            idrow = ids_all[h, pl.ds(g * L, L)]
            pb = (idrow & 1) * D
            for sub in range(2):
                vs = []
                acc = jnp.zeros((L,), jnp.float32)
                for jj in range(8):
                    jl = 8 * sub + jj
                    j = g * L + jl
                    bvec = jnp.full((L,), pb[jl], jnp.int32)
                    jvec = jnp.full((L,), j, jnp.int32)
                    vk = [
                        plsc.load_gather(rb, [jvec, bvec + krow[k]])
                        for k in range(D // L)
                    ]
                    ss = vk[0] * vk[0]
                    for k in range(1, D // L):
                        ss = ss + vk[k] * vk[k]
                    acc = jnp.where(lane == jl, jnp.sum(ss), acc)
                    vs.append(vk)
                fvec = _factor(acc)
                for jj in range(8):
                    jl = 8 * sub + jj
                    j = g * L + jl
                    fs = jnp.full((L,), fvec[jl], jnp.float32)
                    jvec = jnp.full((L,), j, jnp.int32)
                    for k in range(D // L):
                        plsc.store_scatter(
                            tb, [krow[k], jvec], vs[jj][k] * fs
                        )
            return 0

        lax.fori_loop(0, BB // L, g_body, 0)

    pltpu.async_copy(wp_hbm.at[idx_all.at[0]], rows0, rsem)

    def pair_body(hp, _):
        for b in range(2):
            h = 2 * hp + b
            rb, tb = rows[b], tiles[b]
            nb = rows[1 - b]

            @pl.when(h < HIST - 1)
            def _():
                pltpu.async_copy(wp_hbm.at[idx_all.at[h + 1]], nb, rsem)

            pltpu.make_async_copy(
                wp_hbm.at[idx_all.at[h]], rb, rsem
            ).wait()

            @pl.when(h >= 2)
            def _():
                pltpu.make_async_copy(
                    tb, outT_hbm.at[h - 2, :, pl.ds(b0, BB)], osem
                ).wait()

            compute(h, rb, tb)
            pltpu.async_copy(tb, outT_hbm.at[h, :, pl.ds(b0, BB)], osem)
        return 0

    lax.fori_loop(0, HIST // 2, pair_body, 0)
    pltpu.make_async_copy(
        tile0, outT_hbm.at[HIST - 2, :, pl.ds(b0, BB)], osem
    ).wait()
    pltpu.make_async_copy(
        tile1, outT_hbm.at[HIST - 1, :, pl.ds(b0, BB)], osem
    ).wait()


@jax.jit
def _run(idsT, wp):
    mesh = plsc.VectorSubcoreMesh(core_axis_name="c", subcore_axis_name="s")
    return pl.kernel(
        _body,
        out_type=jax.ShapeDtypeStruct((HIST, D, BATCH), jnp.float32),
        mesh=mesh,
        compiler_params=pltpu.CompilerParams(needs_layout_passes=False),
        scratch_types=[
            pltpu.VMEM((HIST, BB), jnp.int32),
            pltpu.VMEM((HIST, BB), jnp.int32),
            pltpu.VMEM((BB, 2 * D), jnp.float32),
            pltpu.VMEM((BB, 2 * D), jnp.float32),
            pltpu.VMEM((D, BB), jnp.float32),
            pltpu.VMEM((D, BB), jnp.float32),
            pltpu.SemaphoreType.DMA,
            pltpu.SemaphoreType.DMA,
        ],
    )(idsT, wp)


def kernel(input_ids, weight):
    idsT = input_ids.T
    wp = weight.reshape(VOCAB // 2, 2 * D)
    outT = _run(idsT, wp)
    return outT.transpose(2, 0, 1)
